# Initial kernel scaffold; baseline (speedup 1.0000x reference)
import jax
import jax.numpy as jnp
from jax.experimental import pallas as pl


def kernel(positions, hash_tables):
    raise NotImplementedError("write your pallas kernel here")


# speedup vs baseline: 1.5215x; 1.5215x over previous
import functools

import jax
import jax.numpy as jnp
import numpy as np
from jax import lax
from jax.experimental import pallas as pl
from jax.experimental.pallas import tpu as pltpu
from jax.experimental.pallas import tpu_sc as plsc

NUM_LEVELS = 16
FEATS = 2
HSIZE = 2 ** 19
MASK = HSIZE - 1
BASE_RES = 16
FINEST_RES = 512
_B_GROWTH = np.exp((np.log(FINEST_RES) - np.log(BASE_RES)) / (NUM_LEVELS - 1))
RES = [int(np.floor(BASE_RES * (_B_GROWTH ** l))) for l in range(NUM_LEVELS)]

C2 = 73856093
C3 = 19349663
PAIRS = [(0, 0), (0, 1), (1, 0), (1, 1)]
OFFQ = [dy * C2 + dz * C3 for (dy, dz) in PAIRS]

NC, NS = 2, 16
NW = NC * NS
CH = 1024
NPAIR = CH * 4
GB = 128
G = NPAIR // GB


def _sc_encode(px, py, pz, tq, resm1, n_points):
    pts_w = n_points // NW
    nch = pts_w // CH
    mesh = plsc.VectorSubcoreMesh(
        core_axis_name="c", subcore_axis_name="s", num_cores=NC, num_subcores=NS)

    @functools.partial(
        pl.kernel,
        mesh=mesh,
        compiler_params=pltpu.CompilerParams(
            needs_layout_passes=False, use_tc_tiling_on_sc=False),
        out_type=jax.ShapeDtypeStruct((n_points * 2 * NUM_LEVELS,), jnp.float32),
        scratch_types=[
            pltpu.VMEM((CH,), jnp.float32),
            pltpu.VMEM((CH,), jnp.float32),
            pltpu.VMEM((CH,), jnp.float32),
            pltpu.VMEM((G, GB), jnp.int32),
            pltpu.VMEM((G, GB, 8), jnp.float32),
            pltpu.VMEM((CH * 2 * NUM_LEVELS,), jnp.float32),
            pltpu.VMEM((NUM_LEVELS,), jnp.float32),
            pltpu.SemaphoreType.DMA,
        ],
    )
    def body(px_hbm, py_hbm, pz_hbm, tq_hbm, resm1_hbm, out_hbm,
             px_v, py_v, pz_v, idx_v, rows_v, out_v, res_v, sem):
        wid = lax.axis_index("c") * NS + lax.axis_index("s")
        iota = lax.iota(jnp.int32, 16)
        dup8 = iota >> 1
        lane_f = iota & 1
        rowpat4 = dup8 * 4
        col_a = lane_f
        col_b = lane_f + 2
        colq = iota * 4
        outpat0 = dup8 * (2 * NUM_LEVELS) + lane_f
        pltpu.sync_copy(resm1_hbm, res_v)

        @pl.loop(0, nch)
        def _chunk(ci):
            base = wid * pts_w + ci * CH
            pltpu.sync_copy(px_hbm.at[pl.ds(base, CH)], px_v)
            pltpu.sync_copy(py_hbm.at[pl.ds(base, CH)], py_v)
            pltpu.sync_copy(pz_hbm.at[pl.ds(base, CH)], pz_v)

            @pl.loop(0, NUM_LEVELS)
            def _level(level):
                lsplat = jnp.full((16,), level, jnp.int32)
                rm1 = plsc.load_gather(res_v, [lsplat])
                loff = level * HSIZE
                outpat = outpat0 + 2 * level

                @pl.loop(0, G)
                def _ia(g):
                    for h16 in range(2):
                        p0 = g * 32 + h16 * 16
                        xx = px_v[pl.ds(p0, 16)]
                        yy = py_v[pl.ds(p0, 16)]
                        zz = pz_v[pl.ds(p0, 16)]
                        xb = (((xx + 1.0) * 0.5) * rm1).astype(jnp.int32)
                        yb = (((yy + 1.0) * 0.5) * rm1).astype(jnp.int32)
                        zb = (((zz + 1.0) * 0.5) * rm1).astype(jnp.int32)
                        h = xb + yb * jnp.int32(C2) + zb * jnp.int32(C3)
                        grow = jnp.full((16,), g, jnp.int32)
                        for q in range(4):
                            v = ((h + jnp.int32(OFFQ[q])) & jnp.int32(MASK)) + loff
                            plsc.store_scatter(
                                idx_v, [grow, colq + (h16 * 64 + q)], v)

                @pl.loop(0, G)
                def _fire(g):
                    pltpu.make_async_copy(
                        tq_hbm.at[idx_v.at[g]], rows_v.at[g], sem).start()

                @pl.loop(0, G)
                def _drain(g):
                    pltpu.make_async_copy(
                        tq_hbm.at[idx_v.at[g]], rows_v.at[g], sem).wait()

                @pl.loop(0, G)
                def _ib(g):
                    rows_g = rows_v.at[g]
                    for s in range(4):
                        pbase = g * 32 + s * 8
                        pid = dup8 + pbase
                        xx = plsc.load_gather(px_v, [pid])
                        yy = plsc.load_gather(py_v, [pid])
                        zz = plsc.load_gather(pz_v, [pid])
                        xs = ((xx + 1.0) * 0.5) * rm1
                        ys = ((yy + 1.0) * 0.5) * rm1
                        zs = ((zz + 1.0) * 0.5) * rm1
                        fx = xs - xs.astype(jnp.int32).astype(jnp.float32)
                        fy = ys - ys.astype(jnp.int32).astype(jnp.float32)
                        fz = zs - zs.astype(jnp.int32).astype(jnp.float32)
                        gx = 1.0 - fx
                        gy = 1.0 - fy
                        gz = 1.0 - fz
                        wyz = [gy * gz, gy * fz, fy * gz, fy * fz]
                        acc = jnp.zeros((16,), jnp.float32)
                        for q in range(4):
                            rowq = rowpat4 + (s * 32 + q)
                            a = plsc.load_gather(rows_g, [rowq, col_a])
                            b = plsc.load_gather(rows_g, [rowq, col_b])
                            acc = acc + (wyz[q] * gx) * a + (wyz[q] * fx) * b
                        oidx = outpat + (g * 1024 + s * 256)
                        plsc.store_scatter(out_v, [oidx], acc)

            pltpu.sync_copy(
                out_v, out_hbm.at[pl.ds(base * (2 * NUM_LEVELS),
                                        CH * 2 * NUM_LEVELS)])

    return body(px, py, pz, tq, resm1)


def kernel(positions, hash_tables):
    bsz, ns, _ = positions.shape
    n = bsz * ns
    pos = positions.reshape(n, 3)
    px = pos[:, 0]
    py = pos[:, 1]
    pz = pos[:, 2]
    tpad = jnp.concatenate([hash_tables, hash_tables[:, :3]], axis=1)
    tq = jnp.concatenate(
        [tpad[:, 0:HSIZE], tpad[:, 1:HSIZE + 1],
         tpad[:, 2:HSIZE + 2], tpad[:, 3:HSIZE + 3]], axis=2)
    tq = tq.reshape(NUM_LEVELS * HSIZE, 4 * FEATS)
    resm1 = jnp.asarray(np.asarray(RES, np.float32) - 1.0)
    out = _sc_encode(px, py, pz, tq, resm1, n)
    return out.reshape(bsz, ns, 2 * NUM_LEVELS)
